# baseline (speedup 1.0000x reference)
import jax
import jax.numpy as jnp
from jax import lax
from jax.experimental import pallas as pl
from jax.experimental.pallas import tpu as pltpu
from jax.experimental.pallas import tpu_sc as plsc

B = 16384
C = 13
V = 100000
D = 8
OUTW = 2 * C * D + C

NC = 2
NSC = 16
NW = NC * NSC
RPW = B // NW
M = 128
NSUB = RPW // M
WBK = 2048
NWB = 49
VP = 100016


def _pad_body(x_ref, o_ref):
    o_ref[...] = x_ref[...]


def _pack_table(W):
    w2 = W.transpose(0, 2, 1).reshape(C * D, V + 1)
    wp = pl.pallas_call(
        _pad_body,
        grid=(C, NWB),
        in_specs=[pl.BlockSpec((D, WBK), lambda c, w: (c, w))],
        out_specs=pl.BlockSpec((D, WBK), lambda c, w: (c, w)),
        out_shape=jax.ShapeDtypeStruct((C * D, VP), jnp.float32),
    )(w2)
    p = wp.reshape(C, D, VP // 16, 16).transpose(0, 2, 3, 1)
    return p.reshape(C * VP, D)


def _body(sT, iT, cT, ws, wi, mb, sb, out,
          vbufs, idxs, tmp, cfbs, cbuf, mv, sv,
          semin, semg, semst, semc):
    cid = lax.axis_index("c")
    sid = lax.axis_index("s")
    wid = sid * NC + cid
    base = wid * RPW

    pltpu.sync_copy(mb, mv)
    pltpu.sync_copy(sb, sv)

    iota = lax.iota(jnp.int32, 16)

    in_handles = []
    for t, vals in enumerate((sT, iT)):
        for c in range(C):
            in_handles.append(pltpu.async_copy(
                vals.at[pl.ds(c * B + base, RPW)], vbufs.at[t * C + c], semin))
    cin_handles = [
        pltpu.async_copy(cT.at[pl.ds(c * B + base, RPW)], cfbs.at[c], semin)
        for c in range(C)
    ]

    def build_idx(t):
        def per_col(c, _c):
            def bld(r, _r):
                raw = vbufs[t * C + c, pl.ds(r * 16, 16)]
                ok = (raw >= 0) & (raw < V)
                idxs[t * C * NSUB + c * NSUB + r // (M // 16),
                     pl.ds((r % (M // 16)) * 16, 16)] = (
                    jnp.where(ok, raw + 1, 0) + c * VP
                )
                return 0
            lax.fori_loop(0, RPW // 16, bld, 0)
            return 0
        lax.fori_loop(0, C, per_col, 0)

    def fire_gathers(t, w):
        return [
            pltpu.async_copy(
                w.at[idxs.at[t * C * NSUB + c * NSUB + q]],
                tmp.at[c, pl.ds(q * M, M)],
                semg,
            )
            for c in range(C) for q in range(NSUB)
        ]

    def fire_stores(toff):
        return [
            pltpu.async_copy(
                tmp.at[c],
                out.at[pl.ds(base, RPW), pl.ds(toff + c * D, D)],
                semst,
            )
            for c in range(C)
        ]

    for h in in_handles[:C]:
        h.wait()
    build_idx(0)
    gs = fire_gathers(0, ws)

    for h in in_handles[C:]:
        h.wait()
    build_idx(1)

    def do_conti(c, _c):
        mvec = mv[pl.ds(c * 16, 16)]
        svec = sv[pl.ds(c * 16, 16)]

        def nrm(k, _k):
            v = (cfbs[c, pl.ds(k * 16, 16)] - mvec) * svec
            plsc.store_scatter(cbuf, [iota + k * 16, iota * 0 + c], v)
            return 0
        lax.fori_loop(0, RPW // 16, nrm, 0)
        return 0

    for h in cin_handles:
        h.wait()
    lax.fori_loop(0, C, do_conti, 0)
    cst = pltpu.async_copy(
        cbuf, out.at[pl.ds(base, RPW), pl.ds(2 * C * D, C)], semc)

    for g in gs:
        g.wait()
    sts = fire_stores(0)
    for s in sts:
        s.wait()

    gs = fire_gathers(1, wi)
    for g in gs:
        g.wait()
    sts = fire_stores(C * D)
    for s in sts:
        s.wait()
    cst.wait()


def kernel(int_vals, str_vals, conti_vals, W_int, W_str, means, variances):
    sT = str_vals.T.reshape(-1)
    iT = int_vals.T.reshape(-1)
    cT = conti_vals.T.reshape(-1)
    ws = _pack_table(W_str)
    wi = _pack_table(W_int)
    mb = jnp.repeat(means, 16)
    sb = jnp.repeat(1.0 / jnp.sqrt(variances), 16)

    kern = pl.kernel(
        _body,
        out_type=jax.ShapeDtypeStruct((B, OUTW), jnp.float32),
        mesh=plsc.VectorSubcoreMesh(core_axis_name="c", subcore_axis_name="s"),
        compiler_params=pltpu.CompilerParams(
            use_tc_tiling_on_sc=False, needs_layout_passes=False
        ),
        scratch_types=[
            pltpu.VMEM((2 * C, RPW), jnp.int32),
            pltpu.VMEM((2 * C * NSUB, M), jnp.int32),
            pltpu.VMEM((C, RPW, D), jnp.float32),
            pltpu.VMEM((C, RPW), jnp.float32),
            pltpu.VMEM((RPW, C), jnp.float32),
            pltpu.VMEM((C * 16,), jnp.float32),
            pltpu.VMEM((C * 16,), jnp.float32),
            pltpu.SemaphoreType.DMA,
            pltpu.SemaphoreType.DMA,
            pltpu.SemaphoreType.DMA,
            pltpu.SemaphoreType.DMA,
        ],
    )
    return kern(sT, iT, cT, ws, wi, mb, sb)


# speedup vs baseline: 1.6340x; 1.6340x over previous
import jax
import jax.numpy as jnp
from jax import lax
from jax.experimental import pallas as pl
from jax.experimental.pallas import tpu as pltpu
from jax.experimental.pallas import tpu_sc as plsc

B = 16384
C = 13
V = 100000
D = 8
OUTW = 2 * C * D + C

NC = 2
NSC = 16
NW = NC * NSC
RPW = B // NW
M = 128
NSUB = RPW // M
WBK = 8192
NWB = 13
VPC = NWB * WBK // 16
VP = VPC * 16


def _pack_body(x_ref, o_ref):
    x = x_ref[...]
    y = jnp.transpose(x.reshape(D, WBK // 16, 16), (1, 2, 0))
    o_ref[...] = y.reshape(WBK // 16, 16 * D)


def _pack_table(W):
    w2 = W.transpose(0, 2, 1).reshape(C * D, V + 1)
    p = pl.pallas_call(
        _pack_body,
        grid=(C, NWB),
        in_specs=[pl.BlockSpec((D, WBK), lambda c, w: (c, w))],
        out_specs=pl.BlockSpec((WBK // 16, 16 * D), lambda c, w: (c * NWB + w, 0)),
        out_shape=jax.ShapeDtypeStruct((C * VPC, 16 * D), jnp.float32),
    )(w2)
    return p.reshape(C * VP, D)


def _body(sT, iT, cT, ws, wi, mb, sb, out,
          vbufs, idxs, tmp, cfbs, cbuf, mv, sv,
          semin, semg, semst, semc):
    cid = lax.axis_index("c")
    sid = lax.axis_index("s")
    wid = sid * NC + cid
    base = wid * RPW

    pltpu.sync_copy(mb, mv)
    pltpu.sync_copy(sb, sv)

    iota = lax.iota(jnp.int32, 16)

    in_handles = []
    for t, vals in enumerate((sT, iT)):
        for c in range(C):
            in_handles.append(pltpu.async_copy(
                vals.at[pl.ds(c * B + base, RPW)], vbufs.at[t * C + c], semin))
    cin_handles = [
        pltpu.async_copy(cT.at[pl.ds(c * B + base, RPW)], cfbs.at[c], semin)
        for c in range(C)
    ]

    def build_idx(t):
        def per_col(c, _c):
            def bld(r, _r):
                raw = vbufs[t * C + c, pl.ds(r * 16, 16)]
                ok = (raw >= 0) & (raw < V)
                idxs[t * C * NSUB + c * NSUB + r // (M // 16),
                     pl.ds((r % (M // 16)) * 16, 16)] = (
                    jnp.where(ok, raw + 1, 0) + c * VP
                )
                return 0
            lax.fori_loop(0, RPW // 16, bld, 0)
            return 0
        lax.fori_loop(0, C, per_col, 0)

    def fire_gathers(t, w):
        return [
            pltpu.async_copy(
                w.at[idxs.at[t * C * NSUB + c * NSUB + q]],
                tmp.at[c, pl.ds(q * M, M)],
                semg,
            )
            for c in range(C) for q in range(NSUB)
        ]

    def fire_stores(toff):
        return [
            pltpu.async_copy(
                tmp.at[c],
                out.at[pl.ds(base, RPW), pl.ds(toff + c * D, D)],
                semst,
            )
            for c in range(C)
        ]

    for h in in_handles[:C]:
        h.wait()
    build_idx(0)
    gs = fire_gathers(0, ws)

    for h in in_handles[C:]:
        h.wait()
    build_idx(1)

    def do_conti(c, _c):
        mvec = mv[pl.ds(c * 16, 16)]
        svec = sv[pl.ds(c * 16, 16)]

        def nrm(k, _k):
            v = (cfbs[c, pl.ds(k * 16, 16)] - mvec) * svec
            plsc.store_scatter(cbuf, [iota + k * 16, iota * 0 + c], v)
            return 0
        lax.fori_loop(0, RPW // 16, nrm, 0)
        return 0

    for h in cin_handles:
        h.wait()
    lax.fori_loop(0, C, do_conti, 0)
    cst = pltpu.async_copy(
        cbuf, out.at[pl.ds(base, RPW), pl.ds(2 * C * D, C)], semc)

    for g in gs:
        g.wait()
    sts = fire_stores(0)
    for s in sts:
        s.wait()

    gs = fire_gathers(1, wi)
    for g in gs:
        g.wait()
    sts = fire_stores(C * D)
    for s in sts:
        s.wait()
    cst.wait()


def kernel(int_vals, str_vals, conti_vals, W_int, W_str, means, variances):
    sT = str_vals.T.reshape(-1)
    iT = int_vals.T.reshape(-1)
    cT = conti_vals.T.reshape(-1)
    ws = _pack_table(W_str)
    wi = _pack_table(W_int)
    mb = jnp.repeat(means, 16)
    sb = jnp.repeat(1.0 / jnp.sqrt(variances), 16)

    kern = pl.kernel(
        _body,
        out_type=jax.ShapeDtypeStruct((B, OUTW), jnp.float32),
        mesh=plsc.VectorSubcoreMesh(core_axis_name="c", subcore_axis_name="s"),
        compiler_params=pltpu.CompilerParams(
            use_tc_tiling_on_sc=False, needs_layout_passes=False
        ),
        scratch_types=[
            pltpu.VMEM((2 * C, RPW), jnp.int32),
            pltpu.VMEM((2 * C * NSUB, M), jnp.int32),
            pltpu.VMEM((C, RPW, D), jnp.float32),
            pltpu.VMEM((C, RPW), jnp.float32),
            pltpu.VMEM((RPW, C), jnp.float32),
            pltpu.VMEM((C * 16,), jnp.float32),
            pltpu.VMEM((C * 16,), jnp.float32),
            pltpu.SemaphoreType.DMA,
            pltpu.SemaphoreType.DMA,
            pltpu.SemaphoreType.DMA,
            pltpu.SemaphoreType.DMA,
        ],
    )
    return kern(sT, iT, cT, ws, wi, mb, sb)
